# Initial kernel scaffold; baseline (speedup 1.0000x reference)
import jax
import jax.numpy as jnp
from jax.experimental import pallas as pl


def kernel(miRNA_embeddings, disease_embeddings, miRNA_index, disease_index, Wm, bm, Wd, bd):
    raise NotImplementedError("write your pallas kernel here")


# speedup vs baseline: 1.7900x; 1.7900x over previous
import functools

import jax
import jax.numpy as jnp
from jax import lax
from jax.experimental import pallas as pl
from jax.experimental.pallas import tpu as pltpu
from jax.experimental.pallas import tpu_sc as plsc

Nm, Nd, F, H, P = 2048, 1024, 512, 256, 4096
NW = 32
BPW = P // NW
PAY_M_ROWS, PAY_D_ROWS = 2112, 1056
PAY_FLAT = PAY_M_ROWS * 1024
DUMP = Nm * Nd
ZCHUNK = PAY_FLAT // 64

_mesh = plsc.VectorSubcoreMesh(core_axis_name="c", subcore_axis_name="s")
_sc_params = pltpu.CompilerParams(needs_layout_passes=False)


def _pos_rank(idx_hbm, out_hbm, n, idx_v, occ_v, pos_v):
    pltpu.sync_copy(idx_hbm, idx_v)

    def zero(i, _):
        occ_v[pl.ds(i * 16, 16)] = jnp.zeros((16,), jnp.int32)
        return 0
    lax.fori_loop(0, n // 16, zero, 0)

    ones = jnp.ones((16,), jnp.int32)

    def scat(i, _):
        plsc.store_scatter(occ_v, [idx_v[pl.ds(i * 16, 16)]], ones)
        return 0
    lax.fori_loop(0, P // 16, scat, 0)

    def csum(i, carry):
        v = occ_v[pl.ds(i * 16, 16)]
        inc = plsc.cumsum(v)
        occ_v[pl.ds(i * 16, 16)] = inc - v + carry
        return carry + jnp.sum(v)
    lax.fori_loop(0, n // 16, csum, jnp.int32(0))

    def gat(i, _):
        pos_v[pl.ds(i * 16, 16)] = plsc.load_gather(occ_v, [idx_v[pl.ds(i * 16, 16)]])
        return 0
    lax.fori_loop(0, P // 16, gat, 0)
    pltpu.sync_copy(pos_v, out_hbm)


_POS_SIG = dict(
    out_type=(jax.ShapeDtypeStruct((P,), jnp.int32),
              jax.ShapeDtypeStruct((P,), jnp.int32)),
    scratch_types=[pltpu.VMEM((P,), jnp.int32),
                   pltpu.VMEM((Nm,), jnp.int32),
                   pltpu.VMEM((P,), jnp.int32)],
)


def _pos_body(mi_hbm, di_hbm, mpos_hbm, dpos_hbm, idx_v, occ_v, pos_v):
    c = lax.axis_index("c")
    s = lax.axis_index("s")

    @pl.when(jnp.logical_and(c == 0, s == 0))
    def _():
        _pos_rank(mi_hbm, mpos_hbm, Nm, idx_v, occ_v, pos_v)

    @pl.when(jnp.logical_and(c == 1, s == 0))
    def _():
        _pos_rank(di_hbm, dpos_hbm, Nd, idx_v, occ_v, pos_v)


_pos_kernel = pl.kernel(_pos_body, mesh=_mesh, compiler_params=_sc_params,
                        **_POS_SIG)


def _proj_body(me_ref, wm_ref, bm_ref, de_ref, wd_ref, bd_ref, pm_ref, pd_ref):
    pm_ref[...] = lax.dot_general(
        me_ref[...], wm_ref[...], (((1,), (1,)), ((), ())),
        preferred_element_type=jnp.float32) + bm_ref[...]
    pd_ref[...] = lax.dot_general(
        de_ref[...], wd_ref[...], (((1,), (1,)), ((), ())),
        preferred_element_type=jnp.float32) + bd_ref[...]


_proj_call = pl.pallas_call(
    _proj_body,
    out_shape=(jax.ShapeDtypeStruct((Nm, H), jnp.float32),
               jax.ShapeDtypeStruct((Nd, H), jnp.float32)),
)


_GATHER_SIG = dict(
    out_type=(jax.ShapeDtypeStruct((P, H), jnp.float32),
              jax.ShapeDtypeStruct((P, H), jnp.float32)),
    scratch_types=[pltpu.VMEM((BPW,), jnp.int32),
                   pltpu.VMEM((BPW, H), jnp.float32),
                   pltpu.SemaphoreType.DMA],
)


def _gather_body(pm_hbm, pd_hbm, mi_hbm, di_hbm, mo_hbm, do_hbm,
                 idx_v, rows_v, sem):
    wid = lax.axis_index("s") * 2 + lax.axis_index("c")
    base = wid * BPW
    pltpu.sync_copy(mi_hbm.at[pl.ds(base, BPW)], idx_v)
    pltpu.async_copy(pm_hbm.at[idx_v], rows_v, sem).wait()
    pltpu.sync_copy(rows_v, mo_hbm.at[pl.ds(base, BPW)])
    pltpu.sync_copy(di_hbm.at[pl.ds(base, BPW)], idx_v)
    pltpu.async_copy(pd_hbm.at[idx_v], rows_v, sem).wait()
    pltpu.sync_copy(rows_v, do_hbm.at[pl.ds(base, BPW)])


_gather_kernel = pl.kernel(_gather_body, mesh=_mesh,
                           compiler_params=_sc_params, **_GATHER_SIG)


def _reward_body(me_ref, de_ref, mpa_ref, dpa_ref, mpc_ref, dpc_ref,
                 rew_ref, cm_ref, cd_ref, cells_sc):
    m = me_ref[...]
    d = de_ref[...]
    num = jnp.sum(m * d, axis=1, keepdims=True)
    den = jnp.sqrt(jnp.sum(m * m, axis=1, keepdims=True)) * \
        jnp.sqrt(jnp.sum(d * d, axis=1, keepdims=True))
    rew_ref[...] = num / den

    cells_sc[...] = mpa_ref[...] * Nd + dpa_ref[...]
    cells_col = mpc_ref[...] * Nd + dpc_ref[...]
    pidx = lax.broadcasted_iota(jnp.int32, (P, 1), 0)
    lane = lax.broadcasted_iota(jnp.int32, (1, 128), 1)

    def body(r, acc):
        crow = cells_sc[pl.ds(r, 1), :]
        eq = jnp.broadcast_to(cells_col, (P, 128)) == jnp.broadcast_to(crow, (P, 128))
        qidx = jnp.broadcast_to(r * 128 + lane, (P, 128))
        return jnp.maximum(acc, jnp.where(eq, qidx, -1))

    acc = lax.fori_loop(0, 32, body, jnp.full((P, 128), -1, jnp.int32))
    maxq = jnp.max(acc, axis=1, keepdims=True)
    win = maxq == pidx
    cm_ref[...] = jnp.where(win, cells_col, DUMP)
    cd_ref[...] = jnp.where(win, dpc_ref[...] * Nm + mpc_ref[...], DUMP)


_reward_call = pl.pallas_call(
    _reward_body,
    out_shape=(jax.ShapeDtypeStruct((P, 1), jnp.float32),
               jax.ShapeDtypeStruct((P, 1), jnp.int32),
               jax.ShapeDtypeStruct((P, 1), jnp.int32)),
    scratch_shapes=[pltpu.VMEM((32, 128), jnp.int32)],
)


_SCATTER_SIG = dict(
    out_type=(jax.ShapeDtypeStruct((PAY_FLAT,), jnp.float32),
              jax.ShapeDtypeStruct((PAY_FLAT,), jnp.float32)),
    scratch_types=[pltpu.VMEM((ZCHUNK,), jnp.float32),
                   pltpu.VMEM((P // 16,), jnp.int32),
                   pltpu.VMEM((P // 16,), jnp.float32),
                   pltpu.SemaphoreType.DMA],
)


def _scatter_body(cm_hbm, cd_hbm, rew_hbm, paym_hbm, payd_hbm,
                  zero_v, cells_v, vals_v, sem):
    c = lax.axis_index("c")
    s = lax.axis_index("s")

    def zbuf(i, _):
        zero_v[pl.ds(i * 16, 16)] = jnp.zeros((16,), jnp.float32)
        return 0
    lax.fori_loop(0, ZCHUNK // 16, zbuf, 0)

    def zdma(i, _):
        off = (s * 4 + i) * ZCHUNK

        @pl.when(c == 0)
        def _():
            pltpu.sync_copy(zero_v, paym_hbm.at[pl.ds(off, ZCHUNK)])

        @pl.when(c == 1)
        def _():
            pltpu.sync_copy(zero_v, payd_hbm.at[pl.ds(off, ZCHUNK)])
        return 0
    lax.fori_loop(0, 4, zdma, 0)

    plsc.subcore_barrier()

    base = s * (P // 16)
    pltpu.sync_copy(rew_hbm.at[pl.ds(base, P // 16)], vals_v)

    @pl.when(c == 0)
    def _():
        pltpu.sync_copy(cm_hbm.at[pl.ds(base, P // 16)], cells_v)
        pltpu.async_copy(vals_v, paym_hbm.at[cells_v], sem).wait()

    @pl.when(c == 1)
    def _():
        pltpu.sync_copy(cd_hbm.at[pl.ds(base, P // 16)], cells_v)
        pltpu.async_copy(vals_v, payd_hbm.at[cells_v], sem).wait()


_scatter_kernel = pl.kernel(_scatter_body, mesh=_mesh,
                            compiler_params=_sc_params, **_SCATTER_SIG)


def _argmax_side(pay, upos, n_rows, n_cols):
    u = jnp.max(upos) + 1
    colid = lax.broadcasted_iota(jnp.int32, (n_rows, n_cols), 1)
    val = jnp.where(colid < u, pay, -jnp.inf)
    rmax = jnp.max(val, axis=1, keepdims=True)
    return jnp.min(jnp.where(val == rmax, colid, n_cols), axis=1, keepdims=True)


def _argmax_body(paym_ref, payd_ref, mpa_ref, dpa_ref, bm_ref, bd_ref):
    bm_ref[...] = _argmax_side(paym_ref[...], dpa_ref[...], Nm, Nd)
    bd_ref[...] = _argmax_side(payd_ref[...], mpa_ref[...], Nd, Nm)


_argmax_call = pl.pallas_call(
    _argmax_body,
    grid=(1,),
    out_shape=(jax.ShapeDtypeStruct((Nm, 1), jnp.int32),
               jax.ShapeDtypeStruct((Nd, 1), jnp.int32)),
    in_specs=[
        pl.BlockSpec((Nm, Nd), lambda i: (0, 0)),
        pl.BlockSpec((Nd, Nm), lambda i: (0, 0)),
        pl.BlockSpec((32, 128), lambda i: (0, 0)),
        pl.BlockSpec((32, 128), lambda i: (0, 0)),
    ],
    out_specs=(pl.BlockSpec((Nm, 1), lambda i: (0, 0)),
               pl.BlockSpec((Nd, 1), lambda i: (0, 0))),
)


_BEST_SIG = dict(
    out_type=(jax.ShapeDtypeStruct((P, H), jnp.float32),
              jax.ShapeDtypeStruct((P, H), jnp.float32)),
    scratch_types=[pltpu.VMEM((Nm,), jnp.int32),
                   pltpu.VMEM((Nd,), jnp.int32),
                   pltpu.VMEM((BPW,), jnp.int32),
                   pltpu.VMEM((BPW,), jnp.int32),
                   pltpu.VMEM((BPW, H), jnp.float32),
                   pltpu.SemaphoreType.DMA],
)


def _best_body(bm_hbm, bd_hbm, mpos_hbm, dpos_hbm, me_hbm, de_hbm,
               bmo_hbm, bdo_hbm, tabm_v, tabd_v, pos_v, sel_v, rows_v, sem):
    wid = lax.axis_index("s") * 2 + lax.axis_index("c")
    base = wid * BPW
    pltpu.sync_copy(bm_hbm, tabm_v)
    pltpu.sync_copy(bd_hbm, tabd_v)

    def run(tab_v, pos_hbm, emb_hbm, out_hbm):
        pltpu.sync_copy(pos_hbm.at[pl.ds(base, BPW)], pos_v)

        def g(i, _):
            sel_v[pl.ds(i * 16, 16)] = plsc.load_gather(
                tab_v, [pos_v[pl.ds(i * 16, 16)]])
            return 0
        lax.fori_loop(0, BPW // 16, g, 0)
        pltpu.async_copy(emb_hbm.at[sel_v], rows_v, sem).wait()
        pltpu.sync_copy(rows_v, out_hbm.at[pl.ds(base, BPW)])

    run(tabm_v, mpos_hbm, me_hbm, bmo_hbm)
    run(tabd_v, dpos_hbm, de_hbm, bdo_hbm)


_best_kernel = pl.kernel(_best_body, mesh=_mesh, compiler_params=_sc_params,
                         **_BEST_SIG)


def _loss_body(me_ref, de_ref, bm_ref, bd_ref, out_ref):
    lm = jnp.mean((me_ref[...] - bm_ref[...]) ** 2)
    ld = jnp.mean((de_ref[...] - bd_ref[...]) ** 2)
    out_ref[0, 0] = (lm + ld) * 0.5


_loss_call = pl.pallas_call(
    _loss_body,
    out_shape=jax.ShapeDtypeStruct((1, 1), jnp.float32),
    out_specs=pl.BlockSpec(memory_space=pltpu.SMEM),
)


def kernel(miRNA_embeddings, disease_embeddings, miRNA_index, disease_index,
           Wm, bm, Wd, bd):
    mi = miRNA_index.astype(jnp.int32)
    di = disease_index.astype(jnp.int32)

    m_pos, d_pos = _pos_kernel(mi, di)
    proj_m, proj_d = _proj_call(
        miRNA_embeddings, Wm, bm.reshape(1, H), disease_embeddings, Wd,
        bd.reshape(1, H))
    m_emb, d_emb = _gather_kernel(proj_m, proj_d, mi, di)

    rew, cm, cd = _reward_call(
        m_emb, d_emb, m_pos.reshape(32, 128), d_pos.reshape(32, 128),
        m_pos.reshape(P, 1), d_pos.reshape(P, 1))

    paym, payd = _scatter_kernel(cm.reshape(P), cd.reshape(P), rew.reshape(P))

    best_m_col, best_d_col = _argmax_call(
        paym.reshape(PAY_M_ROWS, Nd), payd.reshape(PAY_D_ROWS, Nm),
        m_pos.reshape(32, 128), d_pos.reshape(32, 128))

    best_m, best_d = _best_kernel(
        best_m_col.reshape(Nm), best_d_col.reshape(Nd), m_pos, d_pos,
        m_emb, d_emb)

    nash = _loss_call(m_emb, d_emb, best_m, best_d)
    return (nash.reshape(()), best_m, best_d)


# speedup vs baseline: 2.2120x; 1.2358x over previous
import jax
import jax.numpy as jnp
from jax import lax
from jax.experimental import pallas as pl
from jax.experimental.pallas import tpu as pltpu
from jax.experimental.pallas import tpu_sc as plsc

Nm, Nd, F, H, P = 2048, 1024, 512, 256, 4096
NW = 32
BPW = P // NW
RM, RD = 8, 4

MBLK = Nm // 128
DBLK = Nd // 128
M_FLAT = (MBLK + 1) * Nd * 128
D_FLAT = (DBLK + 1) * Nm * 128
DUMP = Nm * Nd
ZWORDS = Nm * Nd
ZCHUNK = ZWORDS // 64

_mesh = plsc.VectorSubcoreMesh(core_axis_name="c", subcore_axis_name="s")
_sc_params = pltpu.CompilerParams(needs_layout_passes=False)


def _pos_rank(idx_hbm, out_hbm, n, idx_v, occ_v, pos_v):
    pltpu.sync_copy(idx_hbm, idx_v)

    def zero(i, _):
        occ_v[pl.ds(i * 16, 16)] = jnp.zeros((16,), jnp.int32)
        return 0
    lax.fori_loop(0, n // 16, zero, 0)

    ones = jnp.ones((16,), jnp.int32)

    def scat(i, _):
        plsc.store_scatter(occ_v, [idx_v[pl.ds(i * 16, 16)]], ones)
        return 0
    lax.fori_loop(0, P // 16, scat, 0)

    def csum(i, carry):
        v = occ_v[pl.ds(i * 16, 16)]
        inc = plsc.cumsum(v)
        occ_v[pl.ds(i * 16, 16)] = inc - v + carry
        return carry + jnp.sum(v)
    lax.fori_loop(0, n // 16, csum, jnp.int32(0))

    def gat(i, _):
        pos_v[pl.ds(i * 16, 16)] = plsc.load_gather(occ_v, [idx_v[pl.ds(i * 16, 16)]])
        return 0
    lax.fori_loop(0, P // 16, gat, 0)
    pltpu.sync_copy(pos_v, out_hbm)


_POS_SIG = dict(
    out_type=(jax.ShapeDtypeStruct((P,), jnp.int32),
              jax.ShapeDtypeStruct((P,), jnp.int32)),
    scratch_types=[pltpu.VMEM((P,), jnp.int32),
                   pltpu.VMEM((Nm,), jnp.int32),
                   pltpu.VMEM((P,), jnp.int32)],
)


def _pos_body(mi_hbm, di_hbm, mpos_hbm, dpos_hbm, idx_v, occ_v, pos_v):
    c = lax.axis_index("c")
    s = lax.axis_index("s")

    @pl.when(jnp.logical_and(c == 0, s == 0))
    def _():
        _pos_rank(mi_hbm, mpos_hbm, Nm, idx_v, occ_v, pos_v)

    @pl.when(jnp.logical_and(c == 1, s == 0))
    def _():
        _pos_rank(di_hbm, dpos_hbm, Nd, idx_v, occ_v, pos_v)


_pos_kernel = pl.kernel(_pos_body, mesh=_mesh, compiler_params=_sc_params,
                        **_POS_SIG)


def _proj_body(me_ref, wm_ref, bm_ref, de_ref, wd_ref, bd_ref, pm_ref, pd_ref):
    pm_ref[...] = lax.dot_general(
        me_ref[...], wm_ref[...], (((1,), (1,)), ((), ())),
        preferred_element_type=jnp.float32) + bm_ref[...]
    pd_ref[...] = lax.dot_general(
        de_ref[...], wd_ref[...], (((1,), (1,)), ((), ())),
        preferred_element_type=jnp.float32) + bd_ref[...]


_proj_call = pl.pallas_call(
    _proj_body,
    out_shape=(jax.ShapeDtypeStruct((Nm, H), jnp.float32),
               jax.ShapeDtypeStruct((Nd, H), jnp.float32)),
)


_GATHER_SIG = dict(
    out_type=(jax.ShapeDtypeStruct((P, H), jnp.float32),
              jax.ShapeDtypeStruct((P, H), jnp.float32),
              jax.ShapeDtypeStruct((RM * Nd, H), jnp.float32),
              jax.ShapeDtypeStruct((RD * Nm, H), jnp.float32)),
    scratch_types=[pltpu.VMEM((BPW,), jnp.int32),
                   pltpu.VMEM((BPW, H), jnp.float32),
                   pltpu.SemaphoreType.DMA],
)


def _gather_body(pm_hbm, pd_hbm, mi_hbm, di_hbm, mo_hbm, do_hbm,
                 repm_hbm, repd_hbm, idx_v, rows_v, sem):
    wid = lax.axis_index("s") * 2 + lax.axis_index("c")
    base = wid * BPW
    pltpu.sync_copy(mi_hbm.at[pl.ds(base, BPW)], idx_v)
    pltpu.async_copy(pm_hbm.at[idx_v], rows_v, sem).wait()
    pltpu.sync_copy(rows_v, mo_hbm.at[pl.ds(base, BPW)])

    @pl.when(wid < Nd // BPW)
    def _():
        for k in range(RM):
            pltpu.sync_copy(rows_v, repm_hbm.at[pl.ds(k * Nd + base, BPW)])

    pltpu.sync_copy(di_hbm.at[pl.ds(base, BPW)], idx_v)
    pltpu.async_copy(pd_hbm.at[idx_v], rows_v, sem).wait()
    pltpu.sync_copy(rows_v, do_hbm.at[pl.ds(base, BPW)])

    @pl.when(wid < Nm // BPW)
    def _():
        for k in range(RD):
            pltpu.sync_copy(rows_v, repd_hbm.at[pl.ds(k * Nm + base, BPW)])


_gather_kernel = pl.kernel(_gather_body, mesh=_mesh,
                           compiler_params=_sc_params, **_GATHER_SIG)


def _reward_body(me_ref, de_ref, mpa_ref, dpa_ref, mpc_ref, dpc_ref,
                 rew_ref, cm_ref, cd_ref, cells_sc, maxq_sc):
    m = me_ref[...]
    d = de_ref[...]
    num = jnp.sum(m * d, axis=1, keepdims=True)
    den = jnp.sqrt(jnp.sum(m * m, axis=1, keepdims=True)) * \
        jnp.sqrt(jnp.sum(d * d, axis=1, keepdims=True))
    rew_ref[...] = num / den

    cells_sc[...] = mpa_ref[...] * Nd + dpa_ref[...]
    cells_col = mpc_ref[...] * Nd + dpc_ref[...]
    qidx = lax.broadcasted_iota(jnp.int32, (P, 1), 0)
    prow = lax.broadcasted_iota(jnp.int32, (32, 128), 0) * 128 + \
        lax.broadcasted_iota(jnp.int32, (32, 128), 1)

    def body(r, _):
        crow = cells_sc[pl.ds(r, 1), :]
        eq = jnp.broadcast_to(cells_col, (P, 128)) == \
            jnp.broadcast_to(crow, (P, 128))
        cand = jnp.where(eq, jnp.broadcast_to(qidx, (P, 128)), -1)
        maxq_sc[pl.ds(r, 1), :] = jnp.max(cand, axis=0, keepdims=True)
        return 0

    lax.fori_loop(0, 32, body, 0)
    win = maxq_sc[...] == prow

    mpa = mpa_ref[...]
    dpa = dpa_ref[...]
    gm = (mpa >> 7) * (Nd * 128) + dpa * 128 + (mpa & 127)
    gd = (dpa >> 7) * (Nm * 128) + mpa * 128 + (dpa & 127)
    cm_ref[...] = jnp.where(win, gm, DUMP)
    cd_ref[...] = jnp.where(win, gd, DUMP)


_reward_call = pl.pallas_call(
    _reward_body,
    out_shape=(jax.ShapeDtypeStruct((P, 1), jnp.float32),
               jax.ShapeDtypeStruct((32, 128), jnp.int32),
               jax.ShapeDtypeStruct((32, 128), jnp.int32)),
    scratch_shapes=[pltpu.VMEM((32, 128), jnp.int32),
                    pltpu.VMEM((32, 128), jnp.int32)],
)


_SCATTER_SIG = dict(
    out_type=(jax.ShapeDtypeStruct((M_FLAT,), jnp.float32),
              jax.ShapeDtypeStruct((D_FLAT,), jnp.float32)),
    scratch_types=[pltpu.VMEM((ZCHUNK,), jnp.float32),
                   pltpu.VMEM((P // 16,), jnp.int32),
                   pltpu.VMEM((P // 16,), jnp.float32),
                   pltpu.SemaphoreType.DMA],
)


def _scatter_body(cm_hbm, cd_hbm, rew_hbm, paym_hbm, payd_hbm,
                  zero_v, cells_v, vals_v, sem):
    c = lax.axis_index("c")
    s = lax.axis_index("s")

    def zbuf(i, _):
        zero_v[pl.ds(i * 16, 16)] = jnp.zeros((16,), jnp.float32)
        return 0
    lax.fori_loop(0, ZCHUNK // 16, zbuf, 0)

    @pl.when(c == 0)
    def _():
        for i in range(4):
            pltpu.make_async_copy(
                zero_v, paym_hbm.at[pl.ds((s * 4 + i) * ZCHUNK, ZCHUNK)],
                sem).start()
        for i in range(4):
            pltpu.make_async_copy(
                zero_v, paym_hbm.at[pl.ds((s * 4 + i) * ZCHUNK, ZCHUNK)],
                sem).wait()

    @pl.when(c == 1)
    def _():
        for i in range(4):
            pltpu.make_async_copy(
                zero_v, payd_hbm.at[pl.ds((s * 4 + i) * ZCHUNK, ZCHUNK)],
                sem).start()
        for i in range(4):
            pltpu.make_async_copy(
                zero_v, payd_hbm.at[pl.ds((s * 4 + i) * ZCHUNK, ZCHUNK)],
                sem).wait()

    plsc.subcore_barrier()

    base = s * (P // 16)
    pltpu.sync_copy(rew_hbm.at[pl.ds(base, P // 16)], vals_v)

    @pl.when(c == 0)
    def _():
        pltpu.sync_copy(cm_hbm.at[pl.ds(base, P // 16)], cells_v)
        pltpu.async_copy(vals_v, paym_hbm.at[cells_v], sem).wait()

    @pl.when(c == 1)
    def _():
        pltpu.sync_copy(cd_hbm.at[pl.ds(base, P // 16)], cells_v)
        pltpu.async_copy(vals_v, payd_hbm.at[cells_v], sem).wait()


_scatter_kernel = pl.kernel(_scatter_body, mesh=_mesh,
                            compiler_params=_sc_params, **_SCATTER_SIG)


def _argmax_body(n_cols, pay_ref, upos_ref, out_ref):
    u = jnp.max(upos_ref[...]) + 1
    chunk = pay_ref[...]
    colid = lax.broadcasted_iota(jnp.int32, (n_cols, 128), 0)
    val = jnp.where(colid < u, chunk, -jnp.inf)
    rmax = jnp.max(val, axis=0, keepdims=True)
    arg = jnp.min(jnp.where(val == rmax, colid, n_cols), axis=0, keepdims=True)
    out_ref[pl.ds(pl.program_id(0), 1), :] = arg


def _mk_argmax(n_rows_blocks, n_cols):
    import functools as _ft
    return pl.pallas_call(
        _ft.partial(_argmax_body, n_cols),
        grid=(n_rows_blocks,),
        out_shape=jax.ShapeDtypeStruct((n_rows_blocks, 128), jnp.int32),
        in_specs=[pl.BlockSpec((n_cols, 128), lambda i: (i, 0)),
                  pl.BlockSpec((32, 128), lambda i: (0, 0))],
        out_specs=pl.BlockSpec((n_rows_blocks, 128), lambda i: (0, 0)),
    )


_argmax_m_call = _mk_argmax(MBLK, Nd)
_argmax_d_call = _mk_argmax(DBLK, Nm)


_BEST_SIG = dict(
    out_type=(jax.ShapeDtypeStruct((P, H), jnp.float32),
              jax.ShapeDtypeStruct((P, H), jnp.float32)),
    scratch_types=[pltpu.VMEM((Nm,), jnp.int32),
                   pltpu.VMEM((Nd,), jnp.int32),
                   pltpu.VMEM((BPW,), jnp.int32),
                   pltpu.VMEM((BPW,), jnp.int32),
                   pltpu.VMEM((BPW, H), jnp.float32),
                   pltpu.SemaphoreType.DMA],
)


def _best_body(bm_hbm, bd_hbm, mpos_hbm, dpos_hbm, repm_hbm, repd_hbm,
               bmo_hbm, bdo_hbm, tabm_v, tabd_v, pos_v, sel_v, rows_v, sem):
    wid = lax.axis_index("s") * 2 + lax.axis_index("c")
    base = wid * BPW
    pltpu.sync_copy(bm_hbm, tabm_v)
    pltpu.sync_copy(bd_hbm, tabd_v)

    def run(tab_v, pos_hbm, rep_hbm, out_hbm, rep_off):
        pltpu.sync_copy(pos_hbm.at[pl.ds(base, BPW)], pos_v)

        def g(i, _):
            sel_v[pl.ds(i * 16, 16)] = plsc.load_gather(
                tab_v, [pos_v[pl.ds(i * 16, 16)]]) + rep_off
            return 0
        lax.fori_loop(0, BPW // 16, g, 0)
        pltpu.async_copy(rep_hbm.at[sel_v], rows_v, sem).wait()
        pltpu.sync_copy(rows_v, out_hbm.at[pl.ds(base, BPW)])

    run(tabm_v, mpos_hbm, repm_hbm, bmo_hbm, (wid % RM) * Nd)
    run(tabd_v, dpos_hbm, repd_hbm, bdo_hbm, (wid % RD) * Nm)


_best_kernel = pl.kernel(_best_body, mesh=_mesh, compiler_params=_sc_params,
                         **_BEST_SIG)


def _loss_body(me_ref, de_ref, bm_ref, bd_ref, out_ref):
    lm = jnp.mean((me_ref[...] - bm_ref[...]) ** 2)
    ld = jnp.mean((de_ref[...] - bd_ref[...]) ** 2)
    out_ref[0, 0] = (lm + ld) * 0.5


_loss_call = pl.pallas_call(
    _loss_body,
    out_shape=jax.ShapeDtypeStruct((1, 1), jnp.float32),
    out_specs=pl.BlockSpec(memory_space=pltpu.SMEM),
)


def kernel(miRNA_embeddings, disease_embeddings, miRNA_index, disease_index,
           Wm, bm, Wd, bd):
    mi = miRNA_index.astype(jnp.int32)
    di = disease_index.astype(jnp.int32)

    m_pos, d_pos = _pos_kernel(mi, di)
    proj_m, proj_d = _proj_call(
        miRNA_embeddings, Wm, bm.reshape(1, H), disease_embeddings, Wd,
        bd.reshape(1, H))
    m_emb, d_emb, rep_m, rep_d = _gather_kernel(proj_m, proj_d, mi, di)

    rew, cm, cd = _reward_call(
        m_emb, d_emb, m_pos.reshape(32, 128), d_pos.reshape(32, 128),
        m_pos.reshape(P, 1), d_pos.reshape(P, 1))

    paym, payd = _scatter_kernel(cm.reshape(P), cd.reshape(P), rew.reshape(P))

    best_m_col = _argmax_m_call(paym.reshape(M_FLAT // 128, 128),
                                d_pos.reshape(32, 128))
    best_d_col = _argmax_d_call(payd.reshape(D_FLAT // 128, 128),
                                m_pos.reshape(32, 128))

    best_m, best_d = _best_kernel(
        best_m_col.reshape(Nm), best_d_col.reshape(Nd), m_pos, d_pos,
        rep_m, rep_d)

    nash = _loss_call(m_emb, d_emb, best_m, best_d)
    return (nash.reshape(()), best_m, best_d)


# speedup vs baseline: 2.2750x; 1.0285x over previous
import jax
import jax.numpy as jnp
from jax import lax
from jax.experimental import pallas as pl
from jax.experimental.pallas import tpu as pltpu
from jax.experimental.pallas import tpu_sc as plsc

Nm, Nd, F, H, P = 2048, 1024, 512, 256, 4096
NW = 32
BPW = P // NW
RM, RD = 8, 4

MBLK = Nm // 128
DBLK = Nd // 128
M_FLAT = (MBLK + 1) * Nd * 128
D_FLAT = (DBLK + 1) * Nm * 128
DUMP = Nm * Nd
ZWORDS = Nm * Nd
ZCHUNK = ZWORDS // 64

_mesh = plsc.VectorSubcoreMesh(core_axis_name="c", subcore_axis_name="s")
_sc_params = pltpu.CompilerParams(needs_layout_passes=False)


NDW = 30
CRANGE = 70656
WPAD = 4224


def _pos_rank(idx_v, out_hbm, n, occ_v, pos_v):
    def zero(i, _):
        occ_v[pl.ds(i * 16, 16)] = jnp.zeros((16,), jnp.int32)
        return 0
    lax.fori_loop(0, n // 16, zero, 0)

    ones = jnp.ones((16,), jnp.int32)

    def scat(i, _):
        plsc.store_scatter(occ_v, [idx_v[pl.ds(i * 16, 16)]], ones)
        return 0
    lax.fori_loop(0, P // 16, scat, 0)

    def csum(i, carry):
        v = occ_v[pl.ds(i * 16, 16)]
        inc = plsc.cumsum(v)
        occ_v[pl.ds(i * 16, 16)] = inc - v + carry
        return carry + jnp.sum(v)
    lax.fori_loop(0, n // 16, csum, jnp.int32(0))

    def gat(i, _):
        pos_v[pl.ds(i * 16, 16)] = plsc.load_gather(occ_v, [idx_v[pl.ds(i * 16, 16)]])
        return 0
    lax.fori_loop(0, P // 16, gat, 0)
    pltpu.sync_copy(pos_v, out_hbm)


def _dedup(rank, mi_v, di_v, own_v, flag_v, tmp_v):
    lane = lax.iota(jnp.int32, 16)
    base = rank * CRANGE
    zeros = jnp.zeros((16,), jnp.int32)
    hugek = jnp.int32(0x40000000)

    def cells16(i):
        return mi_v[pl.ds(i * 16, 16)] * Nd + di_v[pl.ds(i * 16, 16)]

    def p0(i, _):
        lc = cells16(i) - base
        inr = jnp.logical_and(lc >= 0, lc < CRANGE)
        plsc.store_scatter(own_v, [jnp.clip(lc, 0, CRANGE - 1)], zeros,
                           mask=inr)
        return 0
    lax.fori_loop(0, P // 16, p0, 0)

    def p1(i, _):
        lc = cells16(i) - base
        inr = jnp.logical_and(lc >= 0, lc < CRANGE)
        lcc = jnp.clip(lc, 0, CRANGE - 1)
        key = jnp.where(inr, lcc * 16 + lane, hugek + lane)
        sk, sp = plsc.sort_key_val(key, i * 16 + lane)
        sc_ = sk >> 4
        scc = jnp.clip(sc_, 0, CRANGE - 1)
        tmp_v[pl.ds(0, 16)] = sc_
        shifted = plsc.load_gather(tmp_v, [jnp.minimum(lane + 1, 15)])
        last = jnp.logical_or(shifted != sc_, lane == 15)
        valid = sk < hugek
        plsc.store_scatter(own_v, [scc], sp + 1,
                           mask=jnp.logical_and(last, valid))
        return 0
    lax.fori_loop(0, P // 16, p1, 0)

    def p2(i, _):
        lc = cells16(i) - base
        inr = jnp.logical_and(lc >= 0, lc < CRANGE)
        vals = plsc.load_gather(own_v, [jnp.clip(lc, 0, CRANGE - 1)])
        win = jnp.logical_and(inr, vals == i * 16 + lane + 1)
        flag_v[pl.ds(i * 16, 16)] = jnp.where(win, 1, 0)
        return 0
    lax.fori_loop(0, P // 16, p2, 0)


_POS_SIG = dict(
    out_type=(jax.ShapeDtypeStruct((P,), jnp.int32),
              jax.ShapeDtypeStruct((P,), jnp.int32),
              jax.ShapeDtypeStruct((WPAD,), jnp.int32),
              jax.ShapeDtypeStruct((WPAD,), jnp.int32)),
    scratch_types=[pltpu.VMEM((P,), jnp.int32),
                   pltpu.VMEM((P,), jnp.int32),
                   pltpu.VMEM((Nm,), jnp.int32),
                   pltpu.VMEM((P,), jnp.int32),
                   pltpu.VMEM((CRANGE,), jnp.int32),
                   pltpu.VMEM((WPAD,), jnp.int32),
                   pltpu.VMEM((WPAD,), jnp.int32),
                   pltpu.VMEM((16,), jnp.int32),
                   pltpu.VMEM_SHARED((WPAD,), jnp.int32)],
)


def _pos_body(mi_hbm, di_hbm, mpos_hbm, dpos_hbm, win0_hbm, win1_hbm,
              mi_v, di_v, occ_v, pos_v, own_v, flag_v, ident_v, tmp_v,
              shared_win):
    c = lax.axis_index("c")
    s = lax.axis_index("s")
    wid = s * 2 + c
    lane = lax.iota(jnp.int32, 16)

    pltpu.sync_copy(mi_hbm, mi_v)
    pltpu.sync_copy(di_hbm, di_v)

    def init(i, _):
        flag_v[pl.ds(i * 16, 16)] = jnp.zeros((16,), jnp.int32)
        ident_v[pl.ds(i * 16, 16)] = i * 16 + lane
        return 0
    lax.fori_loop(0, WPAD // 16, init, 0)

    @pl.when(s == 0)
    def _():
        pltpu.sync_copy(flag_v, shared_win)

    plsc.subcore_barrier()

    @pl.when(wid == 0)
    def _():
        _pos_rank(mi_v, mpos_hbm, Nm, occ_v, pos_v)

    @pl.when(wid == 1)
    def _():
        _pos_rank(di_v, dpos_hbm, Nd, occ_v, pos_v)

    @pl.when(wid >= 2)
    def _():
        _dedup(wid - 2, mi_v, di_v, own_v, flag_v, tmp_v)
        pltpu.sync_copy(flag_v, shared_win.at[ident_v], add=True)

    plsc.subcore_barrier()

    @pl.when(jnp.logical_and(s == 0, c == 0))
    def _():
        pltpu.sync_copy(shared_win, win0_hbm)

    @pl.when(jnp.logical_and(s == 0, c == 1))
    def _():
        pltpu.sync_copy(shared_win, win1_hbm)


_pos_kernel = pl.kernel(_pos_body, mesh=_mesh, compiler_params=_sc_params,
                        **_POS_SIG)


def _proj_body(me_ref, wm_ref, bm_ref, de_ref, wd_ref, bd_ref, pm_ref, pd_ref):
    pm_ref[...] = lax.dot_general(
        me_ref[...], wm_ref[...], (((1,), (1,)), ((), ())),
        preferred_element_type=jnp.float32) + bm_ref[...]
    pd_ref[...] = lax.dot_general(
        de_ref[...], wd_ref[...], (((1,), (1,)), ((), ())),
        preferred_element_type=jnp.float32) + bd_ref[...]


_proj_call = pl.pallas_call(
    _proj_body,
    out_shape=(jax.ShapeDtypeStruct((Nm, H), jnp.float32),
               jax.ShapeDtypeStruct((Nd, H), jnp.float32)),
)


_GATHER_SIG = dict(
    out_type=(jax.ShapeDtypeStruct((P, H), jnp.float32),
              jax.ShapeDtypeStruct((P, H), jnp.float32),
              jax.ShapeDtypeStruct((RM * Nd, H), jnp.float32),
              jax.ShapeDtypeStruct((RD * Nm, H), jnp.float32)),
    scratch_types=[pltpu.VMEM((BPW,), jnp.int32),
                   pltpu.VMEM((BPW, H), jnp.float32),
                   pltpu.SemaphoreType.DMA],
)


def _gather_body(pm_hbm, pd_hbm, mi_hbm, di_hbm, mo_hbm, do_hbm,
                 repm_hbm, repd_hbm, idx_v, rows_v, sem):
    wid = lax.axis_index("s") * 2 + lax.axis_index("c")
    base = wid * BPW
    pltpu.sync_copy(mi_hbm.at[pl.ds(base, BPW)], idx_v)
    pltpu.async_copy(pm_hbm.at[idx_v], rows_v, sem).wait()
    pltpu.sync_copy(rows_v, mo_hbm.at[pl.ds(base, BPW)])

    @pl.when(wid < Nd // BPW)
    def _():
        for k in range(RM):
            pltpu.sync_copy(rows_v, repm_hbm.at[pl.ds(k * Nd + base, BPW)])

    pltpu.sync_copy(di_hbm.at[pl.ds(base, BPW)], idx_v)
    pltpu.async_copy(pd_hbm.at[idx_v], rows_v, sem).wait()
    pltpu.sync_copy(rows_v, do_hbm.at[pl.ds(base, BPW)])

    @pl.when(wid < Nm // BPW)
    def _():
        for k in range(RD):
            pltpu.sync_copy(rows_v, repd_hbm.at[pl.ds(k * Nm + base, BPW)])


_gather_kernel = pl.kernel(_gather_body, mesh=_mesh,
                           compiler_params=_sc_params, **_GATHER_SIG)


def _reward_body(me_ref, de_ref, mpa_ref, dpa_ref, w0_ref, w1_ref,
                 rew_ref, cm_ref, cd_ref):
    m = me_ref[...]
    d = de_ref[...]
    num = jnp.sum(m * d, axis=1, keepdims=True)
    den = jnp.sqrt(jnp.sum(m * m, axis=1, keepdims=True)) * \
        jnp.sqrt(jnp.sum(d * d, axis=1, keepdims=True))
    rew_ref[...] = num / den

    win = (w0_ref[...] + w1_ref[...]) >= 1
    mpa = mpa_ref[...]
    dpa = dpa_ref[...]
    gm = (mpa >> 7) * (Nd * 128) + dpa * 128 + (mpa & 127)
    gd = (dpa >> 7) * (Nm * 128) + mpa * 128 + (dpa & 127)
    cm_ref[...] = jnp.where(win, gm, DUMP)
    cd_ref[...] = jnp.where(win, gd, DUMP)


_reward_call = pl.pallas_call(
    _reward_body,
    grid=(1,),
    out_shape=(jax.ShapeDtypeStruct((P, 1), jnp.float32),
               jax.ShapeDtypeStruct((32, 128), jnp.int32),
               jax.ShapeDtypeStruct((32, 128), jnp.int32)),
    in_specs=[pl.BlockSpec((P, H), lambda i: (0, 0)),
              pl.BlockSpec((P, H), lambda i: (0, 0)),
              pl.BlockSpec((32, 128), lambda i: (0, 0)),
              pl.BlockSpec((32, 128), lambda i: (0, 0)),
              pl.BlockSpec((32, 128), lambda i: (0, 0)),
              pl.BlockSpec((32, 128), lambda i: (0, 0))],
    out_specs=(pl.BlockSpec((P, 1), lambda i: (0, 0)),
               pl.BlockSpec((32, 128), lambda i: (0, 0)),
               pl.BlockSpec((32, 128), lambda i: (0, 0))),
)


_SCATTER_SIG = dict(
    out_type=(jax.ShapeDtypeStruct((M_FLAT,), jnp.float32),
              jax.ShapeDtypeStruct((D_FLAT,), jnp.float32)),
    scratch_types=[pltpu.VMEM((ZCHUNK,), jnp.float32),
                   pltpu.VMEM((P // 16,), jnp.int32),
                   pltpu.VMEM((P // 16,), jnp.float32),
                   pltpu.SemaphoreType.DMA],
)


def _scatter_body(cm_hbm, cd_hbm, rew_hbm, paym_hbm, payd_hbm,
                  zero_v, cells_v, vals_v, sem):
    c = lax.axis_index("c")
    s = lax.axis_index("s")

    def zbuf(i, _):
        zero_v[pl.ds(i * 16, 16)] = jnp.zeros((16,), jnp.float32)
        return 0
    lax.fori_loop(0, ZCHUNK // 16, zbuf, 0)

    @pl.when(c == 0)
    def _():
        for i in range(4):
            pltpu.make_async_copy(
                zero_v, paym_hbm.at[pl.ds((s * 4 + i) * ZCHUNK, ZCHUNK)],
                sem).start()
        for i in range(4):
            pltpu.make_async_copy(
                zero_v, paym_hbm.at[pl.ds((s * 4 + i) * ZCHUNK, ZCHUNK)],
                sem).wait()

    @pl.when(c == 1)
    def _():
        for i in range(4):
            pltpu.make_async_copy(
                zero_v, payd_hbm.at[pl.ds((s * 4 + i) * ZCHUNK, ZCHUNK)],
                sem).start()
        for i in range(4):
            pltpu.make_async_copy(
                zero_v, payd_hbm.at[pl.ds((s * 4 + i) * ZCHUNK, ZCHUNK)],
                sem).wait()

    plsc.subcore_barrier()

    base = s * (P // 16)
    pltpu.sync_copy(rew_hbm.at[pl.ds(base, P // 16)], vals_v)

    @pl.when(c == 0)
    def _():
        pltpu.sync_copy(cm_hbm.at[pl.ds(base, P // 16)], cells_v)
        pltpu.async_copy(vals_v, paym_hbm.at[cells_v], sem).wait()

    @pl.when(c == 1)
    def _():
        pltpu.sync_copy(cd_hbm.at[pl.ds(base, P // 16)], cells_v)
        pltpu.async_copy(vals_v, payd_hbm.at[cells_v], sem).wait()


_scatter_kernel = pl.kernel(_scatter_body, mesh=_mesh,
                            compiler_params=_sc_params, **_SCATTER_SIG)


def _argmax_body(n_cols, pay_ref, upos_ref, out_ref):
    u = jnp.max(upos_ref[...]) + 1
    chunk = pay_ref[...]
    colid = lax.broadcasted_iota(jnp.int32, (n_cols, 128), 0)
    val = jnp.where(colid < u, chunk, -jnp.inf)
    rmax = jnp.max(val, axis=0, keepdims=True)
    arg = jnp.min(jnp.where(val == rmax, colid, n_cols), axis=0, keepdims=True)
    out_ref[pl.ds(pl.program_id(0), 1), :] = arg


def _mk_argmax(n_rows_blocks, n_cols):
    import functools as _ft
    return pl.pallas_call(
        _ft.partial(_argmax_body, n_cols),
        grid=(n_rows_blocks,),
        out_shape=jax.ShapeDtypeStruct((n_rows_blocks, 128), jnp.int32),
        in_specs=[pl.BlockSpec((n_cols, 128), lambda i: (i, 0)),
                  pl.BlockSpec((32, 128), lambda i: (0, 0))],
        out_specs=pl.BlockSpec((n_rows_blocks, 128), lambda i: (0, 0)),
    )


_argmax_m_call = _mk_argmax(MBLK, Nd)
_argmax_d_call = _mk_argmax(DBLK, Nm)


_BEST_SIG = dict(
    out_type=(jax.ShapeDtypeStruct((P, H), jnp.float32),
              jax.ShapeDtypeStruct((P, H), jnp.float32),
              jax.ShapeDtypeStruct((NW, 16), jnp.float32)),
    scratch_types=[pltpu.VMEM((Nm,), jnp.int32),
                   pltpu.VMEM((Nd,), jnp.int32),
                   pltpu.VMEM((BPW,), jnp.int32),
                   pltpu.VMEM((BPW,), jnp.int32),
                   pltpu.VMEM((BPW, H), jnp.float32),
                   pltpu.VMEM((BPW, H), jnp.float32),
                   pltpu.VMEM((16,), jnp.float32),
                   pltpu.SemaphoreType.DMA],
)


def _best_body(bm_hbm, bd_hbm, mpos_hbm, dpos_hbm, repm_hbm, repd_hbm,
               mo_hbm, do_hbm, bmo_hbm, bdo_hbm, sums_hbm,
               tabm_v, tabd_v, pos_v, sel_v, rows_v, emb_v, acc_v, sem):
    wid = lax.axis_index("s") * 2 + lax.axis_index("c")
    base = wid * BPW
    lane = lax.iota(jnp.int32, 16)
    pltpu.sync_copy(bm_hbm, tabm_v)
    pltpu.sync_copy(bd_hbm, tabd_v)

    def run(tab_v, pos_hbm, rep_hbm, emb_hbm, out_hbm, rep_off):
        pltpu.sync_copy(pos_hbm.at[pl.ds(base, BPW)], pos_v)

        def g(i, _):
            sel_v[pl.ds(i * 16, 16)] = plsc.load_gather(
                tab_v, [pos_v[pl.ds(i * 16, 16)]]) + rep_off
            return 0
        lax.fori_loop(0, BPW // 16, g, 0)
        pltpu.async_copy(rep_hbm.at[sel_v], rows_v, sem).wait()
        pltpu.sync_copy(rows_v, out_hbm.at[pl.ds(base, BPW)])
        pltpu.sync_copy(emb_hbm.at[pl.ds(base, BPW)], emb_v)

        def accum(r, acc):
            for k in range(H // 16):
                dlt = rows_v[r, pl.ds(k * 16, 16)] - emb_v[r, pl.ds(k * 16, 16)]
                acc = acc + dlt * dlt
            return acc
        return lax.fori_loop(0, BPW, accum, jnp.zeros((16,), jnp.float32))

    am = run(tabm_v, mpos_hbm, repm_hbm, mo_hbm, bmo_hbm, (wid % RM) * Nd)
    ad = run(tabd_v, dpos_hbm, repd_hbm, do_hbm, bdo_hbm, (wid % RD) * Nm)
    sm = jnp.sum(am)
    sd = jnp.sum(ad)
    acc_v[pl.ds(0, 16)] = jnp.where(lane == 0, sm,
                                    jnp.where(lane == 1, sd, 0.0))
    pltpu.sync_copy(acc_v, sums_hbm.at[wid])


_best_kernel = pl.kernel(_best_body, mesh=_mesh, compiler_params=_sc_params,
                         **_BEST_SIG)


def kernel(miRNA_embeddings, disease_embeddings, miRNA_index, disease_index,
           Wm, bm, Wd, bd):
    mi = miRNA_index.astype(jnp.int32)
    di = disease_index.astype(jnp.int32)

    m_pos, d_pos, win0, win1 = _pos_kernel(mi, di)
    proj_m, proj_d = _proj_call(
        miRNA_embeddings, Wm, bm.reshape(1, H), disease_embeddings, Wd,
        bd.reshape(1, H))
    m_emb, d_emb, rep_m, rep_d = _gather_kernel(proj_m, proj_d, mi, di)

    rew, cm, cd = _reward_call(
        m_emb, d_emb, m_pos.reshape(32, 128), d_pos.reshape(32, 128),
        win0.reshape(33, 128), win1.reshape(33, 128))

    paym, payd = _scatter_kernel(cm.reshape(P), cd.reshape(P), rew.reshape(P))

    best_m_col = _argmax_m_call(paym.reshape(M_FLAT // 128, 128),
                                d_pos.reshape(32, 128))
    best_d_col = _argmax_d_call(payd.reshape(D_FLAT // 128, 128),
                                m_pos.reshape(32, 128))

    best_m, best_d, sums = _best_kernel(
        best_m_col.reshape(Nm), best_d_col.reshape(Nd), m_pos, d_pos,
        rep_m, rep_d, m_emb, d_emb)

    nash = (jnp.sum(sums[:, 0]) + jnp.sum(sums[:, 1])) / (2.0 * P * H)
    return (nash, best_m, best_d)


# speedup vs baseline: 2.3826x; 1.0473x over previous
import jax
import jax.numpy as jnp
from jax import lax
from jax.experimental import pallas as pl
from jax.experimental.pallas import tpu as pltpu
from jax.experimental.pallas import tpu_sc as plsc

Nm, Nd, F, H, P = 2048, 1024, 512, 256, 4096
NW = 32
BPW = P // NW
RM, RD = 8, 4

MBLK = Nm // 128
DBLK = Nd // 128
M_FLAT = (MBLK + 1) * Nd * 128
D_FLAT = (DBLK + 1) * Nm * 128
DUMP = Nm * Nd
ZWORDS = Nm * Nd
ZCHUNK = 8192

_mesh = plsc.VectorSubcoreMesh(core_axis_name="c", subcore_axis_name="s")
_sc_params = pltpu.CompilerParams(needs_layout_passes=False)


NDW = 30
CRANGE = 70656
WPAD = 4224


def _pos_rank(idx_v, out_hbm, n, occ_v, pos_v):
    def zero(i, _):
        occ_v[pl.ds(i * 16, 16)] = jnp.zeros((16,), jnp.int32)
        return 0
    lax.fori_loop(0, n // 16, zero, 0)

    ones = jnp.ones((16,), jnp.int32)

    def scat(i, _):
        plsc.store_scatter(occ_v, [idx_v[pl.ds(i * 16, 16)]], ones)
        return 0
    lax.fori_loop(0, P // 16, scat, 0)

    def csum(i, carry):
        v = occ_v[pl.ds(i * 16, 16)]
        inc = plsc.cumsum(v)
        occ_v[pl.ds(i * 16, 16)] = inc - v + carry
        return carry + jnp.sum(v)
    lax.fori_loop(0, n // 16, csum, jnp.int32(0))

    def gat(i, _):
        pos_v[pl.ds(i * 16, 16)] = plsc.load_gather(occ_v, [idx_v[pl.ds(i * 16, 16)]])
        return 0
    lax.fori_loop(0, P // 16, gat, 0)
    pltpu.sync_copy(pos_v, out_hbm)


def _dedup(rank, mi_v, di_v, own_v, flag_v, tmp_v):
    lane = lax.iota(jnp.int32, 16)
    base = rank * CRANGE
    zeros = jnp.zeros((16,), jnp.int32)
    hugek = jnp.int32(0x40000000)

    def cells16(i):
        return mi_v[pl.ds(i * 16, 16)] * Nd + di_v[pl.ds(i * 16, 16)]

    def p0(i, _):
        lc = cells16(i) - base
        inr = jnp.logical_and(lc >= 0, lc < CRANGE)
        plsc.store_scatter(own_v, [jnp.clip(lc, 0, CRANGE - 1)], zeros,
                           mask=inr)
        return 0
    lax.fori_loop(0, P // 16, p0, 0)

    def p1(i, _):
        lc = cells16(i) - base
        inr = jnp.logical_and(lc >= 0, lc < CRANGE)
        lcc = jnp.clip(lc, 0, CRANGE - 1)
        key = jnp.where(inr, lcc * 16 + lane, hugek + lane)
        sk, sp = plsc.sort_key_val(key, i * 16 + lane)
        sc_ = sk >> 4
        scc = jnp.clip(sc_, 0, CRANGE - 1)
        tmp_v[pl.ds(0, 16)] = sc_
        shifted = plsc.load_gather(tmp_v, [jnp.minimum(lane + 1, 15)])
        last = jnp.logical_or(shifted != sc_, lane == 15)
        valid = sk < hugek
        plsc.store_scatter(own_v, [scc], sp + 1,
                           mask=jnp.logical_and(last, valid))
        return 0
    lax.fori_loop(0, P // 16, p1, 0)

    def p2(i, _):
        lc = cells16(i) - base
        inr = jnp.logical_and(lc >= 0, lc < CRANGE)
        vals = plsc.load_gather(own_v, [jnp.clip(lc, 0, CRANGE - 1)])
        win = jnp.logical_and(inr, vals == i * 16 + lane + 1)
        flag_v[pl.ds(i * 16, 16)] = jnp.where(win, 1, 0)
        return 0
    lax.fori_loop(0, P // 16, p2, 0)


_POS_SIG = dict(
    out_type=(jax.ShapeDtypeStruct((P,), jnp.int32),
              jax.ShapeDtypeStruct((P,), jnp.int32),
              jax.ShapeDtypeStruct((WPAD,), jnp.int32),
              jax.ShapeDtypeStruct((WPAD,), jnp.int32)),
    scratch_types=[pltpu.VMEM((P,), jnp.int32),
                   pltpu.VMEM((P,), jnp.int32),
                   pltpu.VMEM((Nm,), jnp.int32),
                   pltpu.VMEM((P,), jnp.int32),
                   pltpu.VMEM((CRANGE,), jnp.int32),
                   pltpu.VMEM((WPAD,), jnp.int32),
                   pltpu.VMEM((WPAD,), jnp.int32),
                   pltpu.VMEM((16,), jnp.int32),
                   pltpu.VMEM_SHARED((WPAD,), jnp.int32)],
)


def _pos_body(mi_hbm, di_hbm, mpos_hbm, dpos_hbm, win0_hbm, win1_hbm,
              mi_v, di_v, occ_v, pos_v, own_v, flag_v, ident_v, tmp_v,
              shared_win):
    c = lax.axis_index("c")
    s = lax.axis_index("s")
    wid = s * 2 + c
    lane = lax.iota(jnp.int32, 16)

    pltpu.sync_copy(mi_hbm, mi_v)
    pltpu.sync_copy(di_hbm, di_v)

    def init(i, _):
        flag_v[pl.ds(i * 16, 16)] = jnp.zeros((16,), jnp.int32)
        ident_v[pl.ds(i * 16, 16)] = i * 16 + lane
        return 0
    lax.fori_loop(0, WPAD // 16, init, 0)

    @pl.when(s == 0)
    def _():
        pltpu.sync_copy(flag_v, shared_win)

    plsc.subcore_barrier()

    @pl.when(wid == 0)
    def _():
        _pos_rank(mi_v, mpos_hbm, Nm, occ_v, pos_v)

    @pl.when(wid == 1)
    def _():
        _pos_rank(di_v, dpos_hbm, Nd, occ_v, pos_v)

    @pl.when(wid >= 2)
    def _():
        _dedup(wid - 2, mi_v, di_v, own_v, flag_v, tmp_v)
        pltpu.sync_copy(flag_v, shared_win.at[ident_v], add=True)

    plsc.subcore_barrier()

    @pl.when(jnp.logical_and(s == 0, c == 0))
    def _():
        pltpu.sync_copy(shared_win, win0_hbm)

    @pl.when(jnp.logical_and(s == 0, c == 1))
    def _():
        pltpu.sync_copy(shared_win, win1_hbm)


_pos_kernel = pl.kernel(_pos_body, mesh=_mesh, compiler_params=_sc_params,
                        **_POS_SIG)


def _proj_body(me_ref, wm_ref, bm_ref, de_ref, wd_ref, bd_ref, pm_ref, pd_ref):
    pm_ref[...] = lax.dot_general(
        me_ref[...], wm_ref[...], (((1,), (1,)), ((), ())),
        preferred_element_type=jnp.float32) + bm_ref[...]
    pd_ref[...] = lax.dot_general(
        de_ref[...], wd_ref[...], (((1,), (1,)), ((), ())),
        preferred_element_type=jnp.float32) + bd_ref[...]


_proj_call = pl.pallas_call(
    _proj_body,
    out_shape=(jax.ShapeDtypeStruct((Nm, H), jnp.float32),
               jax.ShapeDtypeStruct((Nd, H), jnp.float32)),
)


_GATHER_SIG = dict(
    out_type=(jax.ShapeDtypeStruct((P, H), jnp.float32),
              jax.ShapeDtypeStruct((P, H), jnp.float32),
              jax.ShapeDtypeStruct((RM * Nd, H), jnp.float32),
              jax.ShapeDtypeStruct((RD * Nm, H), jnp.float32)),
    scratch_types=[pltpu.VMEM((BPW,), jnp.int32),
                   pltpu.VMEM((BPW, H), jnp.float32),
                   pltpu.SemaphoreType.DMA],
)


def _gather_body(pm_hbm, pd_hbm, mi_hbm, di_hbm, mo_hbm, do_hbm,
                 repm_hbm, repd_hbm, idx_v, rows_v, sem):
    wid = lax.axis_index("s") * 2 + lax.axis_index("c")
    base = wid * BPW
    pltpu.sync_copy(mi_hbm.at[pl.ds(base, BPW)], idx_v)
    pltpu.async_copy(pm_hbm.at[idx_v], rows_v, sem).wait()
    pltpu.sync_copy(rows_v, mo_hbm.at[pl.ds(base, BPW)])

    @pl.when(wid < Nd // BPW)
    def _():
        for k in range(RM):
            pltpu.make_async_copy(
                rows_v, repm_hbm.at[pl.ds(k * Nd + base, BPW)], sem).start()
        for k in range(RM):
            pltpu.make_async_copy(
                rows_v, repm_hbm.at[pl.ds(k * Nd + base, BPW)], sem).wait()

    pltpu.sync_copy(di_hbm.at[pl.ds(base, BPW)], idx_v)
    pltpu.async_copy(pd_hbm.at[idx_v], rows_v, sem).wait()
    pltpu.sync_copy(rows_v, do_hbm.at[pl.ds(base, BPW)])

    @pl.when(wid < Nm // BPW)
    def _():
        for k in range(RD):
            pltpu.make_async_copy(
                rows_v, repd_hbm.at[pl.ds(k * Nm + base, BPW)], sem).start()
        for k in range(RD):
            pltpu.make_async_copy(
                rows_v, repd_hbm.at[pl.ds(k * Nm + base, BPW)], sem).wait()


_gather_kernel = pl.kernel(_gather_body, mesh=_mesh,
                           compiler_params=_sc_params, **_GATHER_SIG)


def _reward_body(me_ref, de_ref, mpa_ref, dpa_ref, w0_ref, w1_ref,
                 rew_ref, cm_ref, cd_ref):
    m = me_ref[...]
    d = de_ref[...]
    num = jnp.sum(m * d, axis=1, keepdims=True)
    den = jnp.sqrt(jnp.sum(m * m, axis=1, keepdims=True)) * \
        jnp.sqrt(jnp.sum(d * d, axis=1, keepdims=True))
    rew_ref[...] = num / den

    win = (w0_ref[...] + w1_ref[...]) >= 1
    mpa = mpa_ref[...]
    dpa = dpa_ref[...]
    gm = (mpa >> 7) * (Nd * 128) + dpa * 128 + (mpa & 127)
    gd = (dpa >> 7) * (Nm * 128) + mpa * 128 + (dpa & 127)
    cm_ref[...] = jnp.where(win, gm, DUMP)
    cd_ref[...] = jnp.where(win, gd, DUMP)


_reward_call = pl.pallas_call(
    _reward_body,
    grid=(1,),
    out_shape=(jax.ShapeDtypeStruct((P, 1), jnp.float32),
               jax.ShapeDtypeStruct((32, 128), jnp.int32),
               jax.ShapeDtypeStruct((32, 128), jnp.int32)),
    in_specs=[pl.BlockSpec((P, H), lambda i: (0, 0)),
              pl.BlockSpec((P, H), lambda i: (0, 0)),
              pl.BlockSpec((32, 128), lambda i: (0, 0)),
              pl.BlockSpec((32, 128), lambda i: (0, 0)),
              pl.BlockSpec((32, 128), lambda i: (0, 0)),
              pl.BlockSpec((32, 128), lambda i: (0, 0))],
    out_specs=(pl.BlockSpec((P, 1), lambda i: (0, 0)),
               pl.BlockSpec((32, 128), lambda i: (0, 0)),
               pl.BlockSpec((32, 128), lambda i: (0, 0))),
)


_SCATTER_SIG = dict(
    out_type=(jax.ShapeDtypeStruct((M_FLAT,), jnp.float32),
              jax.ShapeDtypeStruct((D_FLAT,), jnp.float32)),
    scratch_types=[pltpu.VMEM((ZCHUNK,), jnp.float32),
                   pltpu.VMEM((P // 16,), jnp.int32),
                   pltpu.VMEM((P // 16,), jnp.float32),
                   pltpu.SemaphoreType.DMA],
)


def _scatter_body(cm_hbm, cd_hbm, rew_hbm, paym_hbm, payd_hbm,
                  zero_v, cells_v, vals_v, sem):
    c = lax.axis_index("c")
    s = lax.axis_index("s")

    def zbuf(i, _):
        zero_v[pl.ds(i * 16, 16)] = jnp.zeros((16,), jnp.float32)
        return 0
    lax.fori_loop(0, ZCHUNK // 16, zbuf, 0)

    @pl.when(c == 0)
    def _():
        for i in range(16):
            pltpu.make_async_copy(
                zero_v, paym_hbm.at[pl.ds((s * 16 + i) * ZCHUNK, ZCHUNK)],
                sem).start()
        for i in range(16):
            pltpu.make_async_copy(
                zero_v, paym_hbm.at[pl.ds((s * 16 + i) * ZCHUNK, ZCHUNK)],
                sem).wait()

    @pl.when(c == 1)
    def _():
        for i in range(16):
            pltpu.make_async_copy(
                zero_v, payd_hbm.at[pl.ds((s * 16 + i) * ZCHUNK, ZCHUNK)],
                sem).start()
        for i in range(16):
            pltpu.make_async_copy(
                zero_v, payd_hbm.at[pl.ds((s * 16 + i) * ZCHUNK, ZCHUNK)],
                sem).wait()

    plsc.subcore_barrier()

    base = s * (P // 16)
    pltpu.sync_copy(rew_hbm.at[pl.ds(base, P // 16)], vals_v)

    @pl.when(c == 0)
    def _():
        pltpu.sync_copy(cm_hbm.at[pl.ds(base, P // 16)], cells_v)
        pltpu.async_copy(vals_v, paym_hbm.at[cells_v], sem).wait()

    @pl.when(c == 1)
    def _():
        pltpu.sync_copy(cd_hbm.at[pl.ds(base, P // 16)], cells_v)
        pltpu.async_copy(vals_v, payd_hbm.at[cells_v], sem).wait()


_scatter_kernel = pl.kernel(_scatter_body, mesh=_mesh,
                            compiler_params=_sc_params, **_SCATTER_SIG)


def _argmax_body(n_cols, pay_ref, upos_ref, out_ref):
    u = jnp.max(upos_ref[...]) + 1
    chunk = pay_ref[...]
    colid = lax.broadcasted_iota(jnp.int32, (n_cols, 128), 0)
    val = jnp.where(colid < u, chunk, -jnp.inf)
    rmax = jnp.max(val, axis=0, keepdims=True)
    arg = jnp.min(jnp.where(val == rmax, colid, n_cols), axis=0, keepdims=True)
    out_ref[pl.ds(pl.program_id(0), 1), :] = arg


def _mk_argmax(n_rows_blocks, n_cols):
    import functools as _ft
    return pl.pallas_call(
        _ft.partial(_argmax_body, n_cols),
        grid=(n_rows_blocks,),
        out_shape=jax.ShapeDtypeStruct((n_rows_blocks, 128), jnp.int32),
        in_specs=[pl.BlockSpec((n_cols, 128), lambda i: (i, 0)),
                  pl.BlockSpec((32, 128), lambda i: (0, 0))],
        out_specs=pl.BlockSpec((n_rows_blocks, 128), lambda i: (0, 0)),
    )


_argmax_m_call = _mk_argmax(MBLK, Nd)
_argmax_d_call = _mk_argmax(DBLK, Nm)


_BEST_SIG = dict(
    out_type=(jax.ShapeDtypeStruct((P, H), jnp.float32),
              jax.ShapeDtypeStruct((P, H), jnp.float32),
              jax.ShapeDtypeStruct((NW, 16), jnp.float32)),
    scratch_types=[pltpu.VMEM((Nm,), jnp.int32),
                   pltpu.VMEM((Nd,), jnp.int32),
                   pltpu.VMEM((BPW,), jnp.int32),
                   pltpu.VMEM((BPW,), jnp.int32),
                   pltpu.VMEM((BPW, H), jnp.float32),
                   pltpu.VMEM((BPW, H), jnp.float32),
                   pltpu.VMEM((16,), jnp.float32),
                   pltpu.SemaphoreType.DMA],
)


def _best_body(bm_hbm, bd_hbm, mpos_hbm, dpos_hbm, repm_hbm, repd_hbm,
               mo_hbm, do_hbm, bmo_hbm, bdo_hbm, sums_hbm,
               tabm_v, tabd_v, pos_v, sel_v, rows_v, emb_v, acc_v, sem):
    wid = lax.axis_index("s") * 2 + lax.axis_index("c")
    base = wid * BPW
    lane = lax.iota(jnp.int32, 16)
    pltpu.sync_copy(bm_hbm, tabm_v)
    pltpu.sync_copy(bd_hbm, tabd_v)

    def run(tab_v, pos_hbm, rep_hbm, emb_hbm, out_hbm, rep_off):
        pltpu.sync_copy(pos_hbm.at[pl.ds(base, BPW)], pos_v)

        def g(i, _):
            sel_v[pl.ds(i * 16, 16)] = plsc.load_gather(
                tab_v, [pos_v[pl.ds(i * 16, 16)]]) + rep_off
            return 0
        lax.fori_loop(0, BPW // 16, g, 0)
        pltpu.async_copy(rep_hbm.at[sel_v], rows_v, sem).wait()
        pltpu.sync_copy(rows_v, out_hbm.at[pl.ds(base, BPW)])
        pltpu.sync_copy(emb_hbm.at[pl.ds(base, BPW)], emb_v)

        def accum(r, acc):
            for k in range(H // 16):
                dlt = rows_v[r, pl.ds(k * 16, 16)] - emb_v[r, pl.ds(k * 16, 16)]
                acc = acc + dlt * dlt
            return acc
        return lax.fori_loop(0, BPW, accum, jnp.zeros((16,), jnp.float32))

    am = run(tabm_v, mpos_hbm, repm_hbm, mo_hbm, bmo_hbm, (wid % RM) * Nd)
    ad = run(tabd_v, dpos_hbm, repd_hbm, do_hbm, bdo_hbm, (wid % RD) * Nm)
    sm = jnp.sum(am)
    sd = jnp.sum(ad)
    acc_v[pl.ds(0, 16)] = jnp.where(lane == 0, sm,
                                    jnp.where(lane == 1, sd, 0.0))
    pltpu.sync_copy(acc_v, sums_hbm.at[wid])


_best_kernel = pl.kernel(_best_body, mesh=_mesh, compiler_params=_sc_params,
                         **_BEST_SIG)


def kernel(miRNA_embeddings, disease_embeddings, miRNA_index, disease_index,
           Wm, bm, Wd, bd):
    mi = miRNA_index.astype(jnp.int32)
    di = disease_index.astype(jnp.int32)

    m_pos, d_pos, win0, win1 = _pos_kernel(mi, di)
    proj_m, proj_d = _proj_call(
        miRNA_embeddings, Wm, bm.reshape(1, H), disease_embeddings, Wd,
        bd.reshape(1, H))
    m_emb, d_emb, rep_m, rep_d = _gather_kernel(proj_m, proj_d, mi, di)

    rew, cm, cd = _reward_call(
        m_emb, d_emb, m_pos.reshape(32, 128), d_pos.reshape(32, 128),
        win0.reshape(33, 128), win1.reshape(33, 128))

    paym, payd = _scatter_kernel(cm.reshape(P), cd.reshape(P), rew.reshape(P))

    best_m_col = _argmax_m_call(paym.reshape(M_FLAT // 128, 128),
                                d_pos.reshape(32, 128))
    best_d_col = _argmax_d_call(payd.reshape(D_FLAT // 128, 128),
                                m_pos.reshape(32, 128))

    best_m, best_d, sums = _best_kernel(
        best_m_col.reshape(Nm), best_d_col.reshape(Nd), m_pos, d_pos,
        rep_m, rep_d, m_emb, d_emb)

    nash = (jnp.sum(sums[:, 0]) + jnp.sum(sums[:, 1])) / (2.0 * P * H)
    return (nash, best_m, best_d)
